# baseline (speedup 1.0000x reference)
import functools

import jax
import jax.numpy as jnp
from jax import lax
from jax.experimental import pallas as pl
from jax.experimental.pallas import tpu as pltpu
from jax.experimental.pallas import tpu_sc as plsc

NC = 2
NS = 16
LANES = 16
NW = NC * NS
D = 128
CHUNK = 64


def _sc_body(sid_hbm, eid_hbm, stab_hbm, etab_hbm, w_hbm, b_hbm, out_hbm,
             sid_v, eid_v, srows, erows, w_v, b_v, out_v, sem_s, sem_e,
             sem_a, *, b_per_w):
    wid = lax.axis_index("s") * NC + lax.axis_index("c")
    base = wid * b_per_w
    nchunk = b_per_w // CHUNK

    cp_sid = pltpu.async_copy(sid_hbm.at[pl.ds(base, b_per_w)], sid_v, sem_s)
    cp_eid = pltpu.async_copy(eid_hbm.at[pl.ds(base, b_per_w)], eid_v, sem_e)
    cp_w = pltpu.async_copy(w_hbm, w_v, sem_a)
    cp_b = pltpu.async_copy(b_hbm, b_v.at[pl.ds(0, 1)], sem_a)
    cp_sid.wait()
    cp_eid.wait()
    cp_w.wait()
    cp_b.wait()

    ws = [w_v[pl.ds(16 * j, 16)] for j in range(8)]
    we = [w_v[pl.ds(128 + 16 * j, 16)] for j in range(8)]

    def start(c, buf):
        pltpu.async_copy(
            stab_hbm.at[sid_v.at[pl.ds(c * CHUNK, CHUNK)]],
            srows.at[buf], sem_s)
        pltpu.async_copy(
            etab_hbm.at[eid_v.at[pl.ds(c * CHUNK, CHUNK)]],
            erows.at[buf], sem_e)

    def wait(c, buf):
        pltpu.make_async_copy(
            stab_hbm.at[sid_v.at[pl.ds(c * CHUNK, CHUNK)]],
            srows.at[buf], sem_s).wait()
        pltpu.make_async_copy(
            etab_hbm.at[eid_v.at[pl.ds(c * CHUNK, CHUNK)]],
            erows.at[buf], sem_e).wait()

    lane = lax.iota(jnp.int32, LANES)
    first_lane = lane == 0
    perms = [lane ^ (1 << p) for p in range(4)]

    def compute(c, buf):
        @plsc.parallel_loop(0, CHUNK, unroll=3)
        def row(i):
            prods = [srows[buf, i, pl.ds(16 * j, 16)] * ws[j]
                     for j in range(8)]
            prods += [erows[buf, i, pl.ds(16 * j, 16)] * we[j]
                      for j in range(8)]
            while len(prods) > 1:
                prods = [prods[k] + prods[k + 1]
                         for k in range(0, len(prods), 2)]
            acc = prods[0]
            for p in perms:
                acc = acc + jnp.take(acc, p)
            idx = jnp.full((LANES,), c * CHUNK + i, jnp.int32)
            plsc.store_scatter(out_v, [idx], acc, mask=first_lane)

    npair = nchunk // 2
    start(0, 0)

    def pair(cp, _):
        c0 = 2 * cp
        start(c0 + 1, 1)
        wait(c0, 0)
        compute(c0, 0)

        @pl.when(cp + 1 < npair)
        def _():
            start(c0 + 2, 0)

        wait(c0 + 1, 1)
        compute(c0 + 1, 1)
        return 0

    lax.fori_loop(0, npair, pair, 0)

    b_vec = jnp.take(b_v[...], jnp.zeros((LANES,), jnp.int32))

    @plsc.parallel_loop(0, b_per_w, step=LANES, unroll=2)
    def sig(k):
        v = out_v[pl.ds(k, LANES)] + b_vec
        out_v[pl.ds(k, LANES)] = 1.0 / (1.0 + jnp.exp(-v))

    pltpu.sync_copy(out_v, out_hbm.at[pl.ds(base, b_per_w)])


def kernel(student_id, exercise_id, student_table, exercise_table, W, b):
    B = student_id.shape[0]
    b_per_w = B // NW
    sid = student_id.astype(jnp.int32)
    eid = exercise_id.astype(jnp.int32)
    w_flat = W.reshape(-1).astype(jnp.float32)
    b1 = b.astype(jnp.float32)

    mesh = plsc.VectorSubcoreMesh(core_axis_name="c", subcore_axis_name="s")
    run = pl.kernel(
        functools.partial(_sc_body, b_per_w=b_per_w),
        out_type=jax.ShapeDtypeStruct((B,), jnp.float32),
        mesh=mesh,
        compiler_params=pltpu.CompilerParams(needs_layout_passes=False),
        scratch_types=[
            pltpu.VMEM((b_per_w,), jnp.int32),
            pltpu.VMEM((b_per_w,), jnp.int32),
            pltpu.VMEM((2, CHUNK, D), jnp.float32),
            pltpu.VMEM((2, CHUNK, D), jnp.float32),
            pltpu.VMEM((2 * D,), jnp.float32),
            pltpu.VMEM((LANES,), jnp.float32),
            pltpu.VMEM((b_per_w,), jnp.float32),
            pltpu.SemaphoreType.DMA,
            pltpu.SemaphoreType.DMA,
            pltpu.SemaphoreType.DMA,
        ],
    )
    return run(sid, eid, student_table, exercise_table, w_flat, b1)


# speedup vs baseline: 1.0365x; 1.0365x over previous
import functools

import jax
import jax.numpy as jnp
from jax import lax
from jax.experimental import pallas as pl
from jax.experimental.pallas import tpu as pltpu
from jax.experimental.pallas import tpu_sc as plsc

NC = 2
NS = 16
LANES = 16
NW = NC * NS
D = 128
CHUNK = 64


def _sc_body(sid_hbm, eid_hbm, stab_hbm, etab_hbm, w_hbm, b_hbm, out_hbm,
             sid_v, eid_v, srows, erows, w_v, b_v, out_v, sem_s, sem_e,
             sem_a, sem_o, *, b_per_w):
    wid = lax.axis_index("s") * NC + lax.axis_index("c")
    base = wid * b_per_w
    nchunk = b_per_w // CHUNK

    head = 2 * CHUNK
    rest = b_per_w - head
    cp_sid0 = pltpu.async_copy(
        sid_hbm.at[pl.ds(base, head)], sid_v.at[pl.ds(0, head)], sem_s)
    cp_eid0 = pltpu.async_copy(
        eid_hbm.at[pl.ds(base, head)], eid_v.at[pl.ds(0, head)], sem_e)
    cp_sid1 = pltpu.async_copy(
        sid_hbm.at[pl.ds(base + head, rest)],
        sid_v.at[pl.ds(head, rest)], sem_a)
    cp_eid1 = pltpu.async_copy(
        eid_hbm.at[pl.ds(base + head, rest)],
        eid_v.at[pl.ds(head, rest)], sem_a)
    cp_w = pltpu.async_copy(w_hbm, w_v, sem_a)
    cp_b = pltpu.async_copy(b_hbm, b_v.at[pl.ds(0, 1)], sem_a)

    def start(c, buf):
        pltpu.async_copy(
            stab_hbm.at[sid_v.at[pl.ds(c * CHUNK, CHUNK)]],
            srows.at[buf], sem_s)
        pltpu.async_copy(
            etab_hbm.at[eid_v.at[pl.ds(c * CHUNK, CHUNK)]],
            erows.at[buf], sem_e)

    def wait(c, buf):
        pltpu.make_async_copy(
            stab_hbm.at[sid_v.at[pl.ds(c * CHUNK, CHUNK)]],
            srows.at[buf], sem_s).wait()
        pltpu.make_async_copy(
            etab_hbm.at[eid_v.at[pl.ds(c * CHUNK, CHUNK)]],
            erows.at[buf], sem_e).wait()

    cp_sid0.wait()
    cp_eid0.wait()
    start(0, 0)
    cp_sid1.wait()
    cp_eid1.wait()
    cp_w.wait()
    cp_b.wait()

    ws = [w_v[pl.ds(16 * j, 16)] for j in range(8)]
    we = [w_v[pl.ds(128 + 16 * j, 16)] for j in range(8)]
    b_vec = jnp.take(b_v[...], jnp.zeros((LANES,), jnp.int32))

    lane = lax.iota(jnp.int32, LANES)
    first_lane = lane == 0
    perms = [lane ^ (1 << p) for p in range(4)]

    def out_copy(c):
        return pltpu.make_async_copy(
            out_v.at[pl.ds(c * CHUNK, CHUNK)],
            out_hbm.at[pl.ds(base + c * CHUNK, CHUNK)], sem_o)

    def compute(c, buf):
        @plsc.parallel_loop(0, CHUNK, unroll=2)
        def row(i):
            prods = [srows[buf, i, pl.ds(16 * j, 16)] * ws[j]
                     for j in range(8)]
            prods += [erows[buf, i, pl.ds(16 * j, 16)] * we[j]
                      for j in range(8)]
            while len(prods) > 1:
                prods = [prods[k] + prods[k + 1]
                         for k in range(0, len(prods), 2)]
            acc = prods[0]
            for p in perms:
                acc = acc + jnp.take(acc, p)
            idx = jnp.full((LANES,), c * CHUNK + i, jnp.int32)
            plsc.store_scatter(out_v, [idx], acc, mask=first_lane)

        @plsc.parallel_loop(0, CHUNK, step=LANES, unroll=2)
        def sig(k):
            v = out_v[pl.ds(c * CHUNK + k, LANES)] + b_vec
            out_v[pl.ds(c * CHUNK + k, LANES)] = 1.0 / (1.0 + jnp.exp(-v))

        out_copy(c).start()

    npair = nchunk // 2

    def pair(cp, _):
        c0 = 2 * cp
        start(c0 + 1, 1)
        wait(c0, 0)
        compute(c0, 0)

        @pl.when(cp + 1 < npair)
        def _():
            start(c0 + 2, 0)

        wait(c0 + 1, 1)
        compute(c0 + 1, 1)
        return 0

    lax.fori_loop(0, npair, pair, 0)

    def drain(c, _):
        out_copy(c).wait()
        return 0

    lax.fori_loop(0, nchunk, drain, 0)


def kernel(student_id, exercise_id, student_table, exercise_table, W, b):
    B = student_id.shape[0]
    b_per_w = B // NW
    sid = student_id.astype(jnp.int32)
    eid = exercise_id.astype(jnp.int32)
    w_flat = W.reshape(-1).astype(jnp.float32)
    b1 = b.astype(jnp.float32)

    mesh = plsc.VectorSubcoreMesh(core_axis_name="c", subcore_axis_name="s")
    run = pl.kernel(
        functools.partial(_sc_body, b_per_w=b_per_w),
        out_type=jax.ShapeDtypeStruct((B,), jnp.float32),
        mesh=mesh,
        compiler_params=pltpu.CompilerParams(needs_layout_passes=False),
        scratch_types=[
            pltpu.VMEM((b_per_w,), jnp.int32),
            pltpu.VMEM((b_per_w,), jnp.int32),
            pltpu.VMEM((2, CHUNK, D), jnp.float32),
            pltpu.VMEM((2, CHUNK, D), jnp.float32),
            pltpu.VMEM((2 * D,), jnp.float32),
            pltpu.VMEM((LANES,), jnp.float32),
            pltpu.VMEM((b_per_w,), jnp.float32),
            pltpu.SemaphoreType.DMA,
            pltpu.SemaphoreType.DMA,
            pltpu.SemaphoreType.DMA,
            pltpu.SemaphoreType.DMA,
        ],
    )
    return run(sid, eid, student_table, exercise_table, w_flat, b1)


# speedup vs baseline: 1.0479x; 1.0110x over previous
import functools

import jax
import jax.numpy as jnp
from jax import lax
from jax.experimental import pallas as pl
from jax.experimental.pallas import tpu as pltpu
from jax.experimental.pallas import tpu_sc as plsc

NC = 2
NS = 16
LANES = 16
NW = NC * NS
D = 128
CHUNK = 128


def _sc_body(sid_hbm, eid_hbm, stab_hbm, etab_hbm, w_hbm, b_hbm, out_hbm,
             sid_v, eid_v, srows, erows, w_v, b_v, out_v, sem_s, sem_e,
             sem_a, sem_o, *, b_per_w):
    wid = lax.axis_index("s") * NC + lax.axis_index("c")
    base = wid * b_per_w
    nchunk = b_per_w // CHUNK

    head = 2 * CHUNK
    rest = b_per_w - head
    cp_sid0 = pltpu.async_copy(
        sid_hbm.at[pl.ds(base, head)], sid_v.at[pl.ds(0, head)], sem_s)
    cp_eid0 = pltpu.async_copy(
        eid_hbm.at[pl.ds(base, head)], eid_v.at[pl.ds(0, head)], sem_e)
    cp_sid1 = pltpu.async_copy(
        sid_hbm.at[pl.ds(base + head, rest)],
        sid_v.at[pl.ds(head, rest)], sem_a)
    cp_eid1 = pltpu.async_copy(
        eid_hbm.at[pl.ds(base + head, rest)],
        eid_v.at[pl.ds(head, rest)], sem_a)
    cp_w = pltpu.async_copy(w_hbm, w_v, sem_a)
    cp_b = pltpu.async_copy(b_hbm, b_v.at[pl.ds(0, 1)], sem_a)

    def start(c, buf):
        pltpu.async_copy(
            stab_hbm.at[sid_v.at[pl.ds(c * CHUNK, CHUNK)]],
            srows.at[buf], sem_s)
        pltpu.async_copy(
            etab_hbm.at[eid_v.at[pl.ds(c * CHUNK, CHUNK)]],
            erows.at[buf], sem_e)

    def wait(c, buf):
        pltpu.make_async_copy(
            stab_hbm.at[sid_v.at[pl.ds(c * CHUNK, CHUNK)]],
            srows.at[buf], sem_s).wait()
        pltpu.make_async_copy(
            etab_hbm.at[eid_v.at[pl.ds(c * CHUNK, CHUNK)]],
            erows.at[buf], sem_e).wait()

    cp_sid0.wait()
    cp_eid0.wait()
    start(0, 0)
    cp_sid1.wait()
    cp_eid1.wait()
    cp_w.wait()
    cp_b.wait()

    ws = [w_v[pl.ds(16 * j, 16)] for j in range(8)]
    we = [w_v[pl.ds(128 + 16 * j, 16)] for j in range(8)]
    b_vec = jnp.take(b_v[...], jnp.zeros((LANES,), jnp.int32))

    lane = lax.iota(jnp.int32, LANES)
    first_lane = lane == 0
    perms = [lane ^ (1 << p) for p in range(4)]

    def out_copy(c):
        return pltpu.make_async_copy(
            out_v.at[pl.ds(c * CHUNK, CHUNK)],
            out_hbm.at[pl.ds(base + c * CHUNK, CHUNK)], sem_o)

    def compute(c, buf):
        @plsc.parallel_loop(0, CHUNK, unroll=2)
        def row(i):
            prods = [srows[buf, i, pl.ds(16 * j, 16)] * ws[j]
                     for j in range(8)]
            prods += [erows[buf, i, pl.ds(16 * j, 16)] * we[j]
                      for j in range(8)]
            while len(prods) > 1:
                prods = [prods[k] + prods[k + 1]
                         for k in range(0, len(prods), 2)]
            acc = prods[0]
            for p in perms:
                acc = acc + jnp.take(acc, p)
            idx = jnp.full((LANES,), c * CHUNK + i, jnp.int32)
            plsc.store_scatter(out_v, [idx], acc, mask=first_lane)

        @plsc.parallel_loop(0, CHUNK, step=LANES, unroll=2)
        def sig(k):
            v = out_v[pl.ds(c * CHUNK + k, LANES)] + b_vec
            out_v[pl.ds(c * CHUNK + k, LANES)] = 1.0 / (1.0 + jnp.exp(-v))

        out_copy(c).start()

    npair = nchunk // 2

    def pair(cp, _):
        c0 = 2 * cp
        start(c0 + 1, 1)
        wait(c0, 0)
        compute(c0, 0)

        @pl.when(cp + 1 < npair)
        def _():
            start(c0 + 2, 0)

        wait(c0 + 1, 1)
        compute(c0 + 1, 1)
        return 0

    lax.fori_loop(0, npair, pair, 0)

    def drain(c, _):
        out_copy(c).wait()
        return 0

    lax.fori_loop(0, nchunk, drain, 0)


def kernel(student_id, exercise_id, student_table, exercise_table, W, b):
    B = student_id.shape[0]
    b_per_w = B // NW
    sid = student_id.astype(jnp.int32)
    eid = exercise_id.astype(jnp.int32)
    w_flat = W.reshape(-1).astype(jnp.float32)
    b1 = b.astype(jnp.float32)

    mesh = plsc.VectorSubcoreMesh(core_axis_name="c", subcore_axis_name="s")
    run = pl.kernel(
        functools.partial(_sc_body, b_per_w=b_per_w),
        out_type=jax.ShapeDtypeStruct((B,), jnp.float32),
        mesh=mesh,
        compiler_params=pltpu.CompilerParams(needs_layout_passes=False),
        scratch_types=[
            pltpu.VMEM((b_per_w,), jnp.int32),
            pltpu.VMEM((b_per_w,), jnp.int32),
            pltpu.VMEM((2, CHUNK, D), jnp.float32),
            pltpu.VMEM((2, CHUNK, D), jnp.float32),
            pltpu.VMEM((2 * D,), jnp.float32),
            pltpu.VMEM((LANES,), jnp.float32),
            pltpu.VMEM((b_per_w,), jnp.float32),
            pltpu.SemaphoreType.DMA,
            pltpu.SemaphoreType.DMA,
            pltpu.SemaphoreType.DMA,
            pltpu.SemaphoreType.DMA,
        ],
    )
    return run(sid, eid, student_table, exercise_table, w_flat, b1)
